# Initial kernel scaffold; baseline (speedup 1.0000x reference)
import jax
import jax.numpy as jnp
from jax.experimental import pallas as pl


def kernel(target_w, context_w, neg_w, emb_input, emb_output):
    raise NotImplementedError("write your pallas kernel here")


# speedup vs baseline: 3.9973x; 3.9973x over previous
import functools

import jax
import jax.numpy as jnp
from jax import lax
from jax.experimental import pallas as pl
from jax.experimental.pallas import tpu as pltpu
from jax.experimental.pallas import tpu_sc as plsc

B = 16384
D = 64
K = 20
NC = 2
NS = 16
L = 16
NW = NC * NS
BPW = B // NW
BC = 64
NCH = BPW // BC
NIDX_ROWS = BC * K // 128


def _sc_scores(target_w, context_w, neg_w2, emb_input, emb_output):
  mesh = plsc.VectorSubcoreMesh(core_axis_name="c", subcore_axis_name="s")
  f32 = jnp.float32
  i32 = jnp.int32

  @functools.partial(
      pl.kernel,
      out_type=(
          jax.ShapeDtypeStruct((B,), f32),
          jax.ShapeDtypeStruct((B * K,), f32),
      ),
      mesh=mesh,
      compiler_params=pltpu.CompilerParams(
          needs_layout_passes=False, use_tc_tiling_on_sc=False),
      scratch_types=[
          pltpu.VMEM((BC,), i32),
          pltpu.VMEM((BC,), i32),
          pltpu.VMEM((BC * K,), i32),
          pltpu.VMEM((BC, D), f32),
          pltpu.VMEM((BC, D), f32),
          pltpu.VMEM((BC * K, D), f32),
          pltpu.VMEM((BC,), f32),
          pltpu.VMEM((BC * K,), f32),
          pltpu.SemaphoreType.DMA,
      ],
  )
  def sc_kernel(tgt_hbm, ctx_hbm, negi_hbm, tin_hbm, tout_hbm,
                pos_hbm, nego_hbm,
                tidx, cidx, nidx, trows, crows, nrows, posb, negb, sem):
    wid = lax.axis_index("s") * NC + lax.axis_index("c")
    base = wid * BPW
    lane = lax.iota(i32, L)

    def chunk_body(c, _):
      cb = base + c * BC
      pltpu.sync_copy(tgt_hbm.at[pl.ds(cb, BC)], tidx)
      pltpu.sync_copy(ctx_hbm.at[pl.ds(cb, BC)], cidx)
      pltpu.sync_copy(negi_hbm.at[pl.ds(cb * K, BC * K)], nidx)
      cps = [
          pltpu.async_copy(tin_hbm.at[tidx], trows, sem),
          pltpu.async_copy(tout_hbm.at[cidx], crows, sem),
      ]
      for j in range(NIDX_ROWS):
        cps.append(pltpu.async_copy(
            tout_hbm.at[nidx.at[pl.ds(j * 128, 128)]],
            nrows.at[pl.ds(j * 128, 128), :], sem))
      for cp in cps:
        cp.wait()

      def group_body(g, _):
        rows = g * L + lane
        nrow = [rows * K + k for k in range(K)]
        zero = jnp.zeros((L,), f32)

        def dd_body(dd, carry):
          accs = carry
          dv = jnp.broadcast_to(dd, (L,)).astype(i32)
          t = plsc.load_gather(trows, [rows, dv])
          cvec = plsc.load_gather(crows, [rows, dv])
          new = [accs[0] + t * cvec]
          for k in range(K):
            nv = plsc.load_gather(nrows, [nrow[k], dv])
            new.append(accs[k + 1] + t * nv)
          return tuple(new)

        accs = lax.fori_loop(0, D, dd_body, tuple([zero] * (K + 1)))
        posb[pl.ds(g * L, L)] = accs[0]
        for k in range(K):
          negb[pl.ds(k * BC + g * L, L)] = accs[k + 1]
        return 0

      lax.fori_loop(0, BC // L, group_body, 0)
      pltpu.sync_copy(posb, pos_hbm.at[pl.ds(cb, BC)])
      pltpu.sync_copy(negb, nego_hbm.at[pl.ds(cb * K, BC * K)])
      return 0

    lax.fori_loop(0, NCH, chunk_body, 0)

  return sc_kernel(target_w, context_w, neg_w2, emb_input, emb_output)


def _tc_loss(pos2, neg2):
  f32 = jnp.float32

  def tc_body(pos_ref, neg_ref, out_ref):
    p = pos_ref[...]
    n = neg_ref[...]

    def sp(x):
      return jnp.maximum(x, 0.0) + jnp.log1p(jnp.exp(-jnp.abs(x)))

    out_ref[0, 0] = (jnp.sum(sp(-p)) + jnp.sum(sp(n))) / B

  return pl.pallas_call(
      tc_body,
      out_shape=jax.ShapeDtypeStruct((1, 1), f32),
      out_specs=pl.BlockSpec(memory_space=pltpu.SMEM),
  )(pos2, neg2)


def kernel(target_w, context_w, neg_w, emb_input, emb_output):
  neg_w2 = neg_w.astype(jnp.int32).reshape(B * K)
  pos, negs = _sc_scores(target_w.astype(jnp.int32),
                         context_w.astype(jnp.int32),
                         neg_w2, emb_input, emb_output)
  loss = _tc_loss(pos.reshape(B // 128, 128), negs.reshape(B * K // 128, 128))
  return loss[0, 0]


# speedup vs baseline: 4.1938x; 1.0491x over previous
import functools

import jax
import jax.numpy as jnp
from jax import lax
from jax.experimental import pallas as pl
from jax.experimental.pallas import tpu as pltpu
from jax.experimental.pallas import tpu_sc as plsc

B = 16384
D = 64
K = 20
NC = 2
NS = 16
L = 16
NW = NC * NS
BPW = B // NW
BC = 32
BCK = BC * K
NCH = BPW // BC
NPAIR = NCH // 2
NEG_J = BCK // 128


def _sc_scores(target_w, context_w, neg_w_flat, emb_input, emb_output):
  mesh = plsc.VectorSubcoreMesh(core_axis_name="c", subcore_axis_name="s")
  f32 = jnp.float32
  i32 = jnp.int32

  @functools.partial(
      pl.kernel,
      out_type=(
          jax.ShapeDtypeStruct((B,), f32),
          jax.ShapeDtypeStruct((B * K,), f32),
      ),
      mesh=mesh,
      compiler_params=pltpu.CompilerParams(
          needs_layout_passes=False, use_tc_tiling_on_sc=False),
      scratch_types=[
          pltpu.VMEM((BPW,), i32),
          pltpu.VMEM((BPW,), i32),
          pltpu.VMEM((BPW * K,), i32),
          pltpu.VMEM((2, BC, D), f32),
          pltpu.VMEM((2, BC, D), f32),
          pltpu.VMEM((2, BCK, D), f32),
          pltpu.VMEM((BPW,), f32),
          pltpu.VMEM((BPW * K,), f32),
          pltpu.SemaphoreType.DMA,
          pltpu.SemaphoreType.DMA,
      ],
  )
  def sc_kernel(tgt_hbm, ctx_hbm, negi_hbm, tin_hbm, tout_hbm,
                pos_hbm, nego_hbm,
                tidx, cidx, nidx, trows, crows, nrows, posb, negb,
                semA, semB):
    wid = lax.axis_index("s") * NC + lax.axis_index("c")
    base = wid * BPW
    lane = lax.iota(i32, L)

    pltpu.sync_copy(tgt_hbm.at[pl.ds(base, BPW)], tidx)
    pltpu.sync_copy(ctx_hbm.at[pl.ds(base, BPW)], cidx)
    pltpu.sync_copy(negi_hbm.at[pl.ds(base * K, BPW * K)], nidx)

    def fire(c, slot, sem):
      pltpu.async_copy(tin_hbm.at[tidx.at[pl.ds(c * BC, BC)]],
                       trows.at[slot], sem)
      pltpu.async_copy(tout_hbm.at[cidx.at[pl.ds(c * BC, BC)]],
                       crows.at[slot], sem)
      for j in range(NEG_J):
        pltpu.async_copy(
            tout_hbm.at[nidx.at[pl.ds(c * BCK + j * 128, 128)]],
            nrows.at[slot].at[pl.ds(j * 128, 128), :], sem)

    def drain(slot, sem):
      pltpu.make_async_copy(tin_hbm.at[tidx.at[pl.ds(0, BC)]],
                            trows.at[slot], sem).wait()
      pltpu.make_async_copy(tout_hbm.at[cidx.at[pl.ds(0, BC)]],
                            crows.at[slot], sem).wait()
      for j in range(NEG_J):
        pltpu.make_async_copy(
            tout_hbm.at[nidx.at[pl.ds(j * 128, 128)]],
            nrows.at[slot].at[pl.ds(j * 128, 128), :], sem).wait()

    def compute(c, slot):
      tro, cro, nro = trows.at[slot], crows.at[slot], nrows.at[slot]
      for g in range(BC // L):
        rows = g * L + lane
        rowsK = rows * K
        zf = jnp.zeros((L,), f32)

        def jbody(j, carry):
          dv = carry[0]
          accp = carry[1]
          accn = list(carry[2:])
          for u in range(4):
            dvu = dv + u
            t = plsc.load_gather(tro, [rows, dvu])
            cv = plsc.load_gather(cro, [rows, dvu])
            accp = accp + t * cv
            for k in range(K):
              accn[k] = accn[k] + t * plsc.load_gather(nro, [rowsK + k, dvu])
          return (dv + 4, accp, *accn)

        out = lax.fori_loop(0, D // 4, jbody,
                            (jnp.zeros((L,), i32), zf, *([zf] * K)))
        accp = out[1]
        accn = out[2:]
        posb[pl.ds(c * BC + g * L, L)] = accp
        for k in range(K):
          negb[pl.ds(c * BCK + k * BC + g * L, L)] = accn[k]

    fire(0, 0, semA)

    def pair(p, _):
      ca = 2 * p
      fire(ca + 1, 1, semB)
      drain(0, semA)
      compute(ca, 0)

      @pl.when(p < NPAIR - 1)
      def _():
        fire(ca + 2, 0, semA)

      drain(1, semB)
      compute(ca + 1, 1)
      return 0

    lax.fori_loop(0, NPAIR, pair, 0)
    pltpu.sync_copy(posb, pos_hbm.at[pl.ds(base, BPW)])
    pltpu.sync_copy(negb, nego_hbm.at[pl.ds(base * K, BPW * K)])

  return sc_kernel(target_w, context_w, neg_w_flat, emb_input, emb_output)


def _tc_loss(pos2, neg2):
  f32 = jnp.float32

  def tc_body(pos_ref, neg_ref, out_ref):
    p = pos_ref[...]
    n = neg_ref[...]

    def sp(x):
      return jnp.maximum(x, 0.0) + jnp.log1p(jnp.exp(-jnp.abs(x)))

    out_ref[0, 0] = (jnp.sum(sp(-p)) + jnp.sum(sp(n))) / B

  return pl.pallas_call(
      tc_body,
      out_shape=jax.ShapeDtypeStruct((1, 1), f32),
      out_specs=pl.BlockSpec(memory_space=pltpu.SMEM),
  )(pos2, neg2)


def kernel(target_w, context_w, neg_w, emb_input, emb_output):
  neg_w_flat = neg_w.astype(jnp.int32).reshape(B * K)
  pos, negs = _sc_scores(target_w.astype(jnp.int32),
                         context_w.astype(jnp.int32),
                         neg_w_flat, emb_input, emb_output)
  loss = _tc_loss(pos.reshape(B // 128, 128), negs.reshape(B * K // 128, 128))
  return loss[0, 0]


# speedup vs baseline: 4.3125x; 1.0283x over previous
import functools

import jax
import jax.numpy as jnp
from jax import lax
from jax.experimental import pallas as pl
from jax.experimental.pallas import tpu as pltpu
from jax.experimental.pallas import tpu_sc as plsc

B = 16384
D = 64
DP = 128
K = 20
NC = 2
NS = 16
L = 16
NW = NC * NS
BPW = B // NW
BC = 16
BCK = BC * K
NCH = BPW // BC
NPAIR = NCH // 2


def _sc_scores(target_w, context_w, neg_w_flat, emb_input_p, emb_output_p):
  mesh = plsc.VectorSubcoreMesh(core_axis_name="c", subcore_axis_name="s")
  f32 = jnp.float32
  i32 = jnp.int32

  @functools.partial(
      pl.kernel,
      out_type=(
          jax.ShapeDtypeStruct((B,), f32),
          jax.ShapeDtypeStruct((B * K,), f32),
      ),
      mesh=mesh,
      compiler_params=pltpu.CompilerParams(
          needs_layout_passes=False, use_tc_tiling_on_sc=True),
      scratch_types=[
          pltpu.VMEM((BPW,), i32),
          pltpu.VMEM((BPW,), i32),
          pltpu.VMEM((BPW * K,), i32),
          pltpu.VMEM((2, BC, DP), f32),
          pltpu.VMEM((2, BC, DP), f32),
          pltpu.VMEM((2, BCK, DP), f32),
          pltpu.VMEM((BPW,), f32),
          pltpu.VMEM((BPW * K,), f32),
          pltpu.SemaphoreType.DMA,
          pltpu.SemaphoreType.DMA,
      ],
  )
  def sc_kernel(tgt_hbm, ctx_hbm, negi_hbm, tin_hbm, tout_hbm,
                pos_hbm, nego_hbm,
                tidx, cidx, nidx, trows, crows, nrows, posb, negb,
                semA, semB):
    wid = lax.axis_index("s") * NC + lax.axis_index("c")
    base = wid * BPW
    lane = lax.iota(i32, L)

    pltpu.sync_copy(tgt_hbm.at[pl.ds(base, BPW)], tidx)
    pltpu.sync_copy(ctx_hbm.at[pl.ds(base, BPW)], cidx)
    pltpu.sync_copy(negi_hbm.at[pl.ds(base * K, BPW * K)], nidx)

    def fire(c, slot, sem):
      pltpu.async_copy(tin_hbm.at[tidx.at[pl.ds(c * BC, BC)]],
                       trows.at[slot], sem)
      pltpu.async_copy(tout_hbm.at[cidx.at[pl.ds(c * BC, BC)]],
                       crows.at[slot], sem)
      for j in range(0, BCK, 64):
        pltpu.async_copy(
            tout_hbm.at[nidx.at[pl.ds(c * BCK + j, 64)]],
            nrows.at[slot].at[pl.ds(j, 64), :], sem)

    def drain(slot, sem):
      pltpu.make_async_copy(tin_hbm.at[tidx.at[pl.ds(0, BC)]],
                            trows.at[slot], sem).wait()
      pltpu.make_async_copy(tout_hbm.at[cidx.at[pl.ds(0, BC)]],
                            crows.at[slot], sem).wait()
      for j in range(0, BCK, 64):
        pltpu.make_async_copy(
            tout_hbm.at[nidx.at[pl.ds(j, 64)]],
            nrows.at[slot].at[pl.ds(j, 64), :], sem).wait()

    def compute(c, slot):
      tro, cro, nro = trows.at[slot], crows.at[slot], nrows.at[slot]
      rows = lane
      rowsK = lane * K
      zf = jnp.zeros((L,), f32)

      def jbody(j, carry):
        dv = carry[0]
        accp = carry[1]
        accn = list(carry[2:])
        for u in range(4):
          dvu = dv + u
          t = plsc.load_gather(tro, [rows, dvu])
          cv = plsc.load_gather(cro, [rows, dvu])
          accp = accp + t * cv
          for k in range(K):
            accn[k] = accn[k] + t * plsc.load_gather(nro, [rowsK + k, dvu])
        return (dv + 4, accp, *accn)

      out = lax.fori_loop(0, D // 4, jbody,
                          (jnp.zeros((L,), i32), zf, *([zf] * K)))
      accp = out[1]
      accn = out[2:]
      posb[pl.ds(c * BC, L)] = accp
      for k in range(K):
        negb[pl.ds(c * BCK + k * BC, L)] = accn[k]

    fire(0, 0, semA)

    def pair(p, _):
      ca = 2 * p
      fire(ca + 1, 1, semB)
      drain(0, semA)
      compute(ca, 0)

      @pl.when(p < NPAIR - 1)
      def _():
        fire(ca + 2, 0, semA)

      drain(1, semB)
      compute(ca + 1, 1)
      return 0

    lax.fori_loop(0, NPAIR, pair, 0)
    pltpu.sync_copy(posb, pos_hbm.at[pl.ds(base, BPW)])
    pltpu.sync_copy(negb, nego_hbm.at[pl.ds(base * K, BPW * K)])

  return sc_kernel(target_w, context_w, neg_w_flat, emb_input_p, emb_output_p)


def _tc_loss(pos2, neg2):
  f32 = jnp.float32

  def tc_body(pos_ref, neg_ref, out_ref):
    p = pos_ref[...]
    n = neg_ref[...]

    def sp(x):
      return jnp.maximum(x, 0.0) + jnp.log1p(jnp.exp(-jnp.abs(x)))

    out_ref[0, 0] = (jnp.sum(sp(-p)) + jnp.sum(sp(n))) / B

  return pl.pallas_call(
      tc_body,
      out_shape=jax.ShapeDtypeStruct((1, 1), f32),
      out_specs=pl.BlockSpec(memory_space=pltpu.SMEM),
  )(pos2, neg2)


def kernel(target_w, context_w, neg_w, emb_input, emb_output):
  neg_w_flat = neg_w.astype(jnp.int32).reshape(B * K)
  emb_input_p = jnp.pad(emb_input, ((0, 0), (0, DP - D)))
  emb_output_p = jnp.pad(emb_output, ((0, 0), (0, DP - D)))
  pos, negs = _sc_scores(target_w.astype(jnp.int32),
                         context_w.astype(jnp.int32),
                         neg_w_flat, emb_input_p, emb_output_p)
  loss = _tc_loss(pos.reshape(B // 128, 128), negs.reshape(B * K // 128, 128))
  return loss[0, 0]
